# baseline (speedup 1.0000x reference)
import functools

import jax
import jax.numpy as jnp
from jax import lax
from jax.experimental import pallas as pl
from jax.experimental.pallas import tpu as pltpu
from jax.experimental.pallas import tpu_sc as plsc

_B, _P, _L, _M = 4, 8, 1024, 2048
_N = _B * _P
_LAN = 16
_KL = 8
_XW = 80

_mesh = plsc.VectorSubcoreMesh(
    core_axis_name="c", subcore_axis_name="s", num_cores=2, num_subcores=16
)


@functools.partial(
    pl.kernel,
    out_type=jax.ShapeDtypeStruct((_N, 4 * _LAN), jnp.float32),
    mesh=_mesh,
    scratch_types=[
        pltpu.VMEM((_M,), jnp.float32),
        pltpu.VMEM((_M,), jnp.float32),
        pltpu.VMEM((_M,), jnp.float32),
        pltpu.VMEM((_L,), jnp.float32),
        pltpu.VMEM((_L,), jnp.float32),
        pltpu.VMEM((_XW,), jnp.float32),
        pltpu.VMEM((_XW,), jnp.float32),
        pltpu.VMEM((4 * _LAN,), jnp.float32),
        pltpu.VMEM((4 * _LAN,), jnp.float32),
        pltpu.VMEM((_N,), jnp.int32),
        pltpu.VMEM((_N,), jnp.int32),
        pltpu.VMEM((4 * _LAN,), jnp.float32),
        pltpu.SemaphoreType.DMA,
    ],
    compiler_params=pltpu.CompilerParams(needs_layout_passes=False),
)
def _chamfer_sc(x0h, x1h, y0h, y1h, xlh, ylh, out,
                yb0, yb1, wv, xf0, xf1, xw0, xw1, yc0, yc1, xls, yls, ost, sem):
    k = lax.axis_index("s") * 2 + lax.axis_index("c")
    pltpu.sync_copy(xlh, xls)
    pltpu.sync_copy(ylh, yls)
    iot = lax.iota(jnp.int32, _LAN)
    big = jnp.full((_LAN,), 1.0e10, jnp.float32)
    zero = jnp.zeros((_LAN,), jnp.float32)

    def get_len(ref, m):
        c16 = pl.multiple_of((m // _LAN) * _LAN, _LAN)
        return jnp.max(jnp.where((c16 + iot) == m, ref[pl.ds(c16, _LAN)], 0))

    def xmesh(m, carry):
        sx0, sx1 = carry
        nx = get_len(xls, m)
        ny = get_len(yls, m)
        r32 = (nx + _N - 1) // _N
        rl8 = ((r32 + 7) // 8) * 8
        lo = k * rl8
        hi = jnp.minimum(lo + rl8, nx)
        nrows = jnp.maximum(hi - lo, 0)
        wxs = pl.multiple_of(jnp.minimum(lo, _L - 64), 8)
        base = lo - wxs
        ybase = pl.multiple_of(m * _M, 8)
        xoff = pl.multiple_of(m * _L + wxs, 8)
        cp1 = pltpu.async_copy(y0h.at[pl.ds(ybase, _M)], yb0, sem)
        cp2 = pltpu.async_copy(y1h.at[pl.ds(ybase, _M)], yb1, sem)
        cp3 = pltpu.async_copy(x0h.at[pl.ds(xoff, 64)], xw0.at[pl.ds(0, 64)], sem)
        cp4 = pltpu.async_copy(x1h.at[pl.ds(xoff, 64)], xw1.at[pl.ds(0, 64)], sem)
        cp1.wait(); cp2.wait(); cp3.wait(); cp4.wait()
        vb = pl.multiple_of(jnp.minimum((ny // _LAN) * _LAN, _M - _LAN), _LAN)
        mym = (vb + iot) < ny
        yb0[pl.ds(vb, _LAN)] = jnp.where(mym, yb0[pl.ds(vb, _LAN)], 2.0e4)
        yb1[pl.ds(vb, _LAN)] = jnp.where(mym, yb1[pl.ds(vb, _LAN)], 2.0e4)
        ncy = (ny + _LAN - 1) // _LAN

        def wstep(mc, c):
            mb = pl.multiple_of(mc * _LAN, _LAN)
            v0 = yb0[pl.ds(mb, _LAN)]
            v1 = yb1[pl.ds(mb, _LAN)]
            wv[pl.ds(mb, _LAN)] = v0 * v0 + v1 * v1
            return c

        lax.fori_loop(0, ncy, wstep, 0)
        nblk = ((nrows + _KL - 1) // _KL) * 0

        def rblk(b, s):
            rb = pl.multiple_of(base + b * _KL, 8)
            xv0 = xw0[pl.ds(rb, _LAN)]
            xv1 = xw1[pl.ds(rb, _LAN)]
            av = xv0 * xv0 + xv1 * xv1
            t0 = -2.0 * xv0
            t1 = -2.0 * xv1
            bc0 = [jnp.full((_LAN,), t0[i]) for i in range(_KL)]
            bc1 = [jnp.full((_LAN,), t1[i]) for i in range(_KL)]

            def mstep(mc, accs):
                mb = pl.multiple_of(mc * _LAN, _LAN)
                v0 = yb0[pl.ds(mb, _LAN)]
                v1 = yb1[pl.ds(mb, _LAN)]
                wc = wv[pl.ds(mb, _LAN)]
                nacc = []
                for i in range(_KL):
                    g = wc + bc0[i] * v0
                    g = g + bc1[i] * v1
                    nacc.append(jnp.minimum(accs[i], g))
                return tuple(nacc)

            accs = lax.fori_loop(0, ncy, mstep, (big,) * _KL)
            for i in range(_KL):
                rm = jnp.min(accs[i]) + av[i]
                s = s + jnp.where(b * _KL + i < nrows, rm, jnp.float32(0.0))
            return s

        part = lax.fori_loop(0, nblk, rblk, jnp.asarray(0.0, jnp.float32))
        pb = jnp.full((_LAN,), part)
        hit = iot == (m % _LAN)
        g0 = m < _LAN
        sx0 = jnp.where(jnp.logical_and(hit, g0), pb, sx0)
        sx1 = jnp.where(jnp.logical_and(hit, jnp.logical_not(g0)), pb, sx1)
        return (sx0, sx1)

    sx0, sx1 = lax.fori_loop(0, _N, xmesh, (zero, zero))

    def ymesh(m, carry):
        sy0, sy1 = carry
        nx = get_len(xls, m)
        ny = get_len(yls, m)
        c32 = (ny + _N - 1) // _N
        cm16 = ((c32 + _LAN - 1) // _LAN) * _LAN
        clo = k * cm16
        chi = jnp.minimum(clo + cm16, ny)
        clo_s = pl.multiple_of(jnp.minimum(clo, _M - 4 * _LAN), _LAN)
        xbase = pl.multiple_of(m * _L, 8)
        yoff = pl.multiple_of(m * _M + clo_s, 8)
        cp1 = pltpu.async_copy(x0h.at[pl.ds(xbase, _L)], xf0, sem)
        cp2 = pltpu.async_copy(x1h.at[pl.ds(xbase, _L)], xf1, sem)
        cp3 = pltpu.async_copy(y0h.at[pl.ds(yoff, 4 * _LAN)], yc0, sem)
        cp4 = pltpu.async_copy(y1h.at[pl.ds(yoff, 4 * _LAN)], yc1, sem)
        cp1.wait(); cp2.wait(); cp3.wait(); cp4.wait()
        wb = pl.multiple_of(jnp.minimum((nx // _LAN) * _LAN, _L - _LAN), _LAN)
        mxm = (wb + iot) < nx
        xf0[pl.ds(wb, _LAN)] = jnp.where(mxm, xf0[pl.ds(wb, _LAN)], 1.0e4)
        xf1[pl.ds(wb, _LAN)] = jnp.where(mxm, xf1[pl.ds(wb, _LAN)], 1.0e4)
        yy0 = [yc0[pl.ds(c * _LAN, _LAN)] for c in range(4)]
        yy1 = [yc1[pl.ds(c * _LAN, _LAN)] for c in range(4)]
        ncx = (nx + _LAN - 1) // _LAN
        trip = jnp.where(clo < ny, ncx, 0)

        def rstep(rc, maccs):
            rb = pl.multiple_of(rc * _LAN, _LAN)
            xv0 = xf0[pl.ds(rb, _LAN)]
            xv1 = xf1[pl.ds(rb, _LAN)]
            av = xv0 * xv0 + xv1 * xv1
            t0 = -2.0 * xv0
            t1 = -2.0 * xv1
            nacc = list(maccs)
            for i in range(_LAN):
                s0 = jnp.full((_LAN,), t0[i])
                s1 = jnp.full((_LAN,), t1[i])
                ab = jnp.full((_LAN,), av[i])
                for c in range(4):
                    g = ab + s0 * yy0[c]
                    g = g + s1 * yy1[c]
                    nacc[c] = jnp.minimum(nacc[c], g)
            return tuple(nacc)

        maccs = lax.fori_loop(0, trip, rstep, (big,) * 4)
        sumv = zero
        for c in range(4):
            wc = yy0[c] * yy0[c] + yy1[c] * yy1[c]
            colv = clo_s + c * _LAN + iot
            valid = jnp.logical_and(colv >= clo, colv < chi)
            sumv = sumv + jnp.where(valid, maccs[c] + wc, jnp.float32(0.0))
        part = jnp.sum(sumv)
        pb = jnp.full((_LAN,), part)
        hit = iot == (m % _LAN)
        g0 = m < _LAN
        sy0 = jnp.where(jnp.logical_and(hit, g0), pb, sy0)
        sy1 = jnp.where(jnp.logical_and(hit, jnp.logical_not(g0)), pb, sy1)
        return (sy0, sy1)

    sy0, sy1 = lax.fori_loop(0, _N, ymesh, (zero, zero))

    ost[pl.ds(0, _LAN)] = sx0
    ost[pl.ds(_LAN, _LAN)] = sx1
    ost[pl.ds(2 * _LAN, _LAN)] = sy0
    ost[pl.ds(3 * _LAN, _LAN)] = sy1
    pltpu.sync_copy(ost, out.at[k])


def kernel(boundaries, edgemaps, boundary_lengths, edgemaps_len):
    bx = boundaries[..., 0].reshape(_N * _L)
    by = boundaries[..., 1].reshape(_N * _L)
    ex = edgemaps[..., 0].reshape(_N * _M)
    ey = edgemaps[..., 1].reshape(_N * _M)
    xl = boundary_lengths.reshape(_N).astype(jnp.int32)
    yl = edgemaps_len.reshape(_N).astype(jnp.int32)
    o = _chamfer_sc(bx, by, ex, ey, xl, yl)
    sx = o[:, : 2 * _LAN].reshape(_N, 2 * _LAN).sum(axis=0)
    sy = o[:, 2 * _LAN :].sum(axis=0)
    xlf = xl.astype(jnp.float32)
    ylf = yl.astype(jnp.float32)
    loss = sx / xlf + sy / ylf
    return loss.reshape(_B, _P).mean(axis=1) * 10.0


# speedup vs baseline: 1.9888x; 1.9888x over previous
import functools

import jax
import jax.numpy as jnp
from jax import lax
from jax.experimental import pallas as pl
from jax.experimental.pallas import tpu as pltpu
from jax.experimental.pallas import tpu_sc as plsc

_B, _P, _L, _M = 4, 8, 1024, 2048
_N = _B * _P
_LAN = 16
_KL = 8

_mesh = plsc.VectorSubcoreMesh(
    core_axis_name="c", subcore_axis_name="s", num_cores=2, num_subcores=16
)


@functools.partial(
    pl.kernel,
    out_type=jax.ShapeDtypeStruct((_N, 4 * _LAN), jnp.float32),
    mesh=_mesh,
    scratch_types=[
        pltpu.VMEM((_M + _LAN,), jnp.float32),
        pltpu.VMEM((_M + _LAN,), jnp.float32),
        pltpu.VMEM((_L + _LAN,), jnp.float32),
        pltpu.VMEM((_L + _LAN,), jnp.float32),
        pltpu.VMEM((_N,), jnp.int32),
        pltpu.VMEM((_N,), jnp.int32),
        pltpu.VMEM((4 * _LAN,), jnp.float32),
        pltpu.SemaphoreType.DMA,
    ],
    compiler_params=pltpu.CompilerParams(needs_layout_passes=False),
)
def _chamfer_sc(x0h, x1h, y0h, y1h, xlh, ylh, out,
                yb0, yb1, xf0, xf1, xls, yls, ost, sem):
    k = lax.axis_index("s") * 2 + lax.axis_index("c")
    pltpu.sync_copy(xlh, xls)
    pltpu.sync_copy(ylh, yls)
    iot = lax.iota(jnp.int32, _LAN)
    big = jnp.full((_LAN,), 1.0e10, jnp.float32)
    zero = jnp.zeros((_LAN,), jnp.float32)
    fzero = jnp.float32(0.0)

    def get_len(ref, m):
        c16 = pl.multiple_of((m // _LAN) * _LAN, _LAN)
        return jnp.max(jnp.where((c16 + iot) == m, ref[pl.ds(c16, _LAN)], 0))

    def mesh_step(m, carry):
        sx0, sx1, sy0, sy1 = carry
        nx = get_len(xls, m)
        ny = get_len(yls, m)
        xbase = pl.multiple_of(m * _L, 8)
        ybase = pl.multiple_of(m * _M, 8)
        cp1 = pltpu.async_copy(y0h.at[pl.ds(ybase, _M)], yb0.at[pl.ds(0, _M)], sem)
        cp2 = pltpu.async_copy(y1h.at[pl.ds(ybase, _M)], yb1.at[pl.ds(0, _M)], sem)
        cp3 = pltpu.async_copy(x0h.at[pl.ds(xbase, _L)], xf0.at[pl.ds(0, _L)], sem)
        cp4 = pltpu.async_copy(x1h.at[pl.ds(xbase, _L)], xf1.at[pl.ds(0, _L)], sem)
        cp1.wait(); cp2.wait(); cp3.wait(); cp4.wait()

        vb = pl.multiple_of(jnp.minimum((ny // _LAN) * _LAN, _M - _LAN), _LAN)
        mym = (vb + iot) < ny
        yb0[pl.ds(vb, _LAN)] = jnp.where(mym, yb0[pl.ds(vb, _LAN)], 2.0e4)
        yb1[pl.ds(vb, _LAN)] = jnp.where(mym, yb1[pl.ds(vb, _LAN)], 2.0e4)
        wb = pl.multiple_of(jnp.minimum((nx // _LAN) * _LAN, _L - _LAN), _LAN)
        mxm = (wb + iot) < nx
        xf0[pl.ds(wb, _LAN)] = jnp.where(mxm, xf0[pl.ds(wb, _LAN)], 1.0e4)
        xf1[pl.ds(wb, _LAN)] = jnp.where(mxm, xf1[pl.ds(wb, _LAN)], 1.0e4)

        ncy = (ny + _LAN - 1) // _LAN
        ncx = (nx + _LAN - 1) // _LAN

        r32 = (nx + _N - 1) // _N
        rl8 = ((r32 + _KL - 1) // _KL) * _KL
        lo = k * rl8
        hi = jnp.minimum(lo + rl8, nx)
        nrows = jnp.maximum(hi - lo, 0)
        nblk = (nrows + _KL - 1) // _KL

        def rblk(b, s):
            rb = pl.multiple_of(lo + b * _KL, _KL)
            xv0 = xf0[pl.ds(rb, _LAN)]
            xv1 = xf1[pl.ds(rb, _LAN)]
            av = xv0 * xv0 + xv1 * xv1
            t0 = -2.0 * xv0
            t1 = -2.0 * xv1
            bc0 = [jnp.full((_LAN,), t0[i]) for i in range(_KL)]
            bc1 = [jnp.full((_LAN,), t1[i]) for i in range(_KL)]

            def mstep(mc, accs):
                mb = pl.multiple_of(mc * _LAN, _LAN)
                v0 = yb0[pl.ds(mb, _LAN)]
                v1 = yb1[pl.ds(mb, _LAN)]
                wc = v0 * v0 + v1 * v1
                nacc = []
                for i in range(_KL):
                    g = wc + bc0[i] * v0
                    g = g + bc1[i] * v1
                    nacc.append(jnp.minimum(accs[i], g))
                return tuple(nacc)

            accs = lax.fori_loop(0, ncy, mstep, (big,) * _KL)
            for i in range(_KL):
                rm = jnp.min(accs[i]) + av[i]
                s = s + jnp.where(b * _KL + i < nrows, rm, fzero)
            return s

        partx = lax.fori_loop(0, nblk, rblk, fzero)

        c32 = (ny + _N - 1) // _N
        cm8 = ((c32 + _KL - 1) // _KL) * _KL
        clo = k * cm8
        chi = jnp.minimum(clo + cm8, ny)
        ncols = jnp.maximum(chi - clo, 0)
        ncb = (ncols + _KL - 1) // _KL

        def cblk(b, s):
            cb = pl.multiple_of(clo + b * _KL, _KL)
            yv0 = yb0[pl.ds(cb, _LAN)]
            yv1 = yb1[pl.ds(cb, _LAN)]
            wv = yv0 * yv0 + yv1 * yv1
            t0 = -2.0 * yv0
            t1 = -2.0 * yv1
            bc0 = [jnp.full((_LAN,), t0[i]) for i in range(_KL)]
            bc1 = [jnp.full((_LAN,), t1[i]) for i in range(_KL)]

            def rstep(rc, accs):
                rv = pl.multiple_of(rc * _LAN, _LAN)
                u0 = xf0[pl.ds(rv, _LAN)]
                u1 = xf1[pl.ds(rv, _LAN)]
                ac = u0 * u0 + u1 * u1
                nacc = []
                for i in range(_KL):
                    g = ac + bc0[i] * u0
                    g = g + bc1[i] * u1
                    nacc.append(jnp.minimum(accs[i], g))
                return tuple(nacc)

            accs = lax.fori_loop(0, ncx, rstep, (big,) * _KL)
            for i in range(_KL):
                cmn = jnp.min(accs[i]) + wv[i]
                s = s + jnp.where(b * _KL + i < ncols, cmn, fzero)
            return s

        party = lax.fori_loop(0, ncb, cblk, fzero)

        hit = iot == (m % _LAN)
        g0 = m < _LAN
        hit0 = jnp.logical_and(hit, g0)
        hit1 = jnp.logical_and(hit, jnp.logical_not(g0))
        pxb = jnp.full((_LAN,), partx)
        pyb = jnp.full((_LAN,), party)
        sx0 = jnp.where(hit0, pxb, sx0)
        sx1 = jnp.where(hit1, pxb, sx1)
        sy0 = jnp.where(hit0, pyb, sy0)
        sy1 = jnp.where(hit1, pyb, sy1)
        return (sx0, sx1, sy0, sy1)

    sx0, sx1, sy0, sy1 = lax.fori_loop(0, _N, mesh_step, (zero, zero, zero, zero))

    ost[pl.ds(0, _LAN)] = sx0
    ost[pl.ds(_LAN, _LAN)] = sx1
    ost[pl.ds(2 * _LAN, _LAN)] = sy0
    ost[pl.ds(3 * _LAN, _LAN)] = sy1
    pltpu.sync_copy(ost, out.at[k])


def kernel(boundaries, edgemaps, boundary_lengths, edgemaps_len):
    bx = boundaries[..., 0].reshape(_N * _L)
    by = boundaries[..., 1].reshape(_N * _L)
    ex = edgemaps[..., 0].reshape(_N * _M)
    ey = edgemaps[..., 1].reshape(_N * _M)
    xl = boundary_lengths.reshape(_N).astype(jnp.int32)
    yl = edgemaps_len.reshape(_N).astype(jnp.int32)
    o = _chamfer_sc(bx, by, ex, ey, xl, yl)
    sx = o[:, : 2 * _LAN].sum(axis=0)
    sy = o[:, 2 * _LAN :].sum(axis=0)
    xlf = xl.astype(jnp.float32)
    ylf = yl.astype(jnp.float32)
    loss = sx / xlf + sy / ylf
    return loss.reshape(_B, _P).mean(axis=1) * 10.0


# speedup vs baseline: 2.4000x; 1.2068x over previous
import functools

import jax
import jax.numpy as jnp
from jax import lax
from jax.experimental import pallas as pl
from jax.experimental.pallas import tpu as pltpu
from jax.experimental.pallas import tpu_sc as plsc

_B, _P, _L, _M = 4, 8, 1024, 2048
_N = _B * _P
_LAN = 16
_KL = 8

_mesh = plsc.VectorSubcoreMesh(
    core_axis_name="c", subcore_axis_name="s", num_cores=2, num_subcores=16
)


@functools.partial(
    pl.kernel,
    out_type=jax.ShapeDtypeStruct((_N, 4 * _LAN), jnp.float32),
    mesh=_mesh,
    scratch_types=[
        pltpu.VMEM((_M + 2 * _LAN,), jnp.float32),
        pltpu.VMEM((_M + 2 * _LAN,), jnp.float32),
        pltpu.VMEM((_L + 2 * _LAN,), jnp.float32),
        pltpu.VMEM((_L + 2 * _LAN,), jnp.float32),
        pltpu.VMEM((_M + 2 * _LAN,), jnp.float32),
        pltpu.VMEM((_M + 2 * _LAN,), jnp.float32),
        pltpu.VMEM((_L + 2 * _LAN,), jnp.float32),
        pltpu.VMEM((_L + 2 * _LAN,), jnp.float32),
        pltpu.VMEM((_N,), jnp.int32),
        pltpu.VMEM((_N,), jnp.int32),
        pltpu.VMEM((4 * _LAN,), jnp.float32),
        pltpu.SemaphoreType.DMA,
        pltpu.SemaphoreType.DMA,
    ],
    compiler_params=pltpu.CompilerParams(needs_layout_passes=False),
)
def _chamfer_sc(x0h, x1h, y0h, y1h, xlh, ylh, out,
                ya0, ya1, xa0, xa1, yb0, yb1, xb0, xb1, xls, yls, ost,
                semA, semB):
    k = lax.axis_index("s") * 2 + lax.axis_index("c")
    pltpu.sync_copy(xlh, xls)
    pltpu.sync_copy(ylh, yls)
    iot = lax.iota(jnp.int32, _LAN)
    big = jnp.full((_LAN,), 1.0e10, jnp.float32)
    zero = jnp.zeros((_LAN,), jnp.float32)
    fzero = jnp.float32(0.0)

    def get_len(ref, m):
        c16 = pl.multiple_of((m // _LAN) * _LAN, _LAN)
        return jnp.max(jnp.where((c16 + iot) == m, ref[pl.ds(c16, _LAN)], 0))

    def issue(m, y0s, y1s, x0s, x1s, sem):
        xbase = pl.multiple_of(m * _L, 8)
        ybase = pl.multiple_of(m * _M, 8)
        c1 = pltpu.async_copy(y0h.at[pl.ds(ybase, _M)], y0s.at[pl.ds(0, _M)], sem)
        c2 = pltpu.async_copy(y1h.at[pl.ds(ybase, _M)], y1s.at[pl.ds(0, _M)], sem)
        c3 = pltpu.async_copy(x0h.at[pl.ds(xbase, _L)], x0s.at[pl.ds(0, _L)], sem)
        c4 = pltpu.async_copy(x1h.at[pl.ds(xbase, _L)], x1s.at[pl.ds(0, _L)], sem)
        return (c1, c2, c3, c4)

    def process(m, carry, y0s, y1s, x0s, x1s):
        sx0, sx1, sy0, sy1 = carry
        nx = get_len(xls, m)
        ny = get_len(yls, m)

        vb = pl.multiple_of(jnp.minimum((ny // _LAN) * _LAN, _M - 2 * _LAN), _LAN)
        for off in (0, _LAN):
            mym = (vb + off + iot) < ny
            y0s[pl.ds(vb + off, _LAN)] = jnp.where(mym, y0s[pl.ds(vb + off, _LAN)], 2.0e4)
            y1s[pl.ds(vb + off, _LAN)] = jnp.where(mym, y1s[pl.ds(vb + off, _LAN)], 2.0e4)
        wb = pl.multiple_of(jnp.minimum((nx // _LAN) * _LAN, _L - 2 * _LAN), _LAN)
        for off in (0, _LAN):
            mxm = (wb + off + iot) < nx
            x0s[pl.ds(wb + off, _LAN)] = jnp.where(mxm, x0s[pl.ds(wb + off, _LAN)], 1.0e4)
            x1s[pl.ds(wb + off, _LAN)] = jnp.where(mxm, x1s[pl.ds(wb + off, _LAN)], 1.0e4)

        ncy2 = (ny + 2 * _LAN - 1) // (2 * _LAN)
        ncx2 = (nx + 2 * _LAN - 1) // (2 * _LAN)

        r32 = (nx + _N - 1) // _N
        rl8 = ((r32 + _KL - 1) // _KL) * _KL
        lo = k * rl8
        hi = jnp.minimum(lo + rl8, nx)
        nrows = jnp.maximum(hi - lo, 0)
        nblk = (nrows + _KL - 1) // _KL

        def rblk(b, s):
            rb = pl.multiple_of(lo + b * _KL, _KL)
            xv0 = x0s[pl.ds(rb, _LAN)]
            xv1 = x1s[pl.ds(rb, _LAN)]
            av = xv0 * xv0 + xv1 * xv1
            t0 = -2.0 * xv0
            t1 = -2.0 * xv1
            bc0 = [jnp.full((_LAN,), t0[i]) for i in range(_KL)]
            bc1 = [jnp.full((_LAN,), t1[i]) for i in range(_KL)]

            def mstep(mc, accs):
                nacc = list(accs)
                for half in (0, 1):
                    mb = pl.multiple_of(mc * 2 * _LAN + half * _LAN, _LAN)
                    v0 = y0s[pl.ds(mb, _LAN)]
                    v1 = y1s[pl.ds(mb, _LAN)]
                    wc = v0 * v0 + v1 * v1
                    for i in range(_KL):
                        g = wc + bc0[i] * v0
                        g = g + bc1[i] * v1
                        nacc[i] = jnp.minimum(nacc[i], g)
                return tuple(nacc)

            accs = lax.fori_loop(0, ncy2, mstep, (big,) * _KL)
            for i in range(_KL):
                rm = jnp.min(accs[i]) + av[i]
                s = s + jnp.where(b * _KL + i < nrows, rm, fzero)
            return s

        partx = lax.fori_loop(0, nblk, rblk, fzero)

        c32 = (ny + _N - 1) // _N
        cm8 = ((c32 + _KL - 1) // _KL) * _KL
        clo = k * cm8
        chi = jnp.minimum(clo + cm8, ny)
        ncols = jnp.maximum(chi - clo, 0)
        ncb = (ncols + _KL - 1) // _KL

        def cblk(b, s):
            cb = pl.multiple_of(clo + b * _KL, _KL)
            yv0 = y0s[pl.ds(cb, _LAN)]
            yv1 = y1s[pl.ds(cb, _LAN)]
            wv = yv0 * yv0 + yv1 * yv1
            t0 = -2.0 * yv0
            t1 = -2.0 * yv1
            bc0 = [jnp.full((_LAN,), t0[i]) for i in range(_KL)]
            bc1 = [jnp.full((_LAN,), t1[i]) for i in range(_KL)]

            def rstep(rc, accs):
                nacc = list(accs)
                for half in (0, 1):
                    rv = pl.multiple_of(rc * 2 * _LAN + half * _LAN, _LAN)
                    u0 = x0s[pl.ds(rv, _LAN)]
                    u1 = x1s[pl.ds(rv, _LAN)]
                    ac = u0 * u0 + u1 * u1
                    for i in range(_KL):
                        g = ac + bc0[i] * u0
                        g = g + bc1[i] * u1
                        nacc[i] = jnp.minimum(nacc[i], g)
                return tuple(nacc)

            accs = lax.fori_loop(0, ncx2, rstep, (big,) * _KL)
            for i in range(_KL):
                cmn = jnp.min(accs[i]) + wv[i]
                s = s + jnp.where(b * _KL + i < ncols, cmn, fzero)
            return s

        party = lax.fori_loop(0, ncb, cblk, fzero)

        hit = iot == (m % _LAN)
        g0 = m < _LAN
        hit0 = jnp.logical_and(hit, g0)
        hit1 = jnp.logical_and(hit, jnp.logical_not(g0))
        pxb = jnp.full((_LAN,), partx)
        pyb = jnp.full((_LAN,), party)
        sx0 = jnp.where(hit0, pxb, sx0)
        sx1 = jnp.where(hit1, pxb, sx1)
        sy0 = jnp.where(hit0, pyb, sy0)
        sy1 = jnp.where(hit1, pyb, sy1)
        return (sx0, sx1, sy0, sy1)

    issue(0, ya0, ya1, xa0, xa1, semA)

    def wait_slot(y0s, y1s, x0s, x1s, sem):
        pltpu.make_async_copy(y0h.at[pl.ds(0, _M)], y0s.at[pl.ds(0, _M)], sem).wait()
        pltpu.make_async_copy(y1h.at[pl.ds(0, _M)], y1s.at[pl.ds(0, _M)], sem).wait()
        pltpu.make_async_copy(x0h.at[pl.ds(0, _L)], x0s.at[pl.ds(0, _L)], sem).wait()
        pltpu.make_async_copy(x1h.at[pl.ds(0, _L)], x1s.at[pl.ds(0, _L)], sem).wait()

    def pair_step2(mm, carry):
        m0 = mm * 2
        m1 = m0 + 1
        m2 = jnp.minimum(m0 + 2, _N - 1)
        issue(m1, yb0, yb1, xb0, xb1, semB)
        wait_slot(ya0, ya1, xa0, xa1, semA)
        carry = process(m0, carry, ya0, ya1, xa0, xa1)
        issue(m2, ya0, ya1, xa0, xa1, semA)
        wait_slot(yb0, yb1, xb0, xb1, semB)
        carry = process(m1, carry, yb0, yb1, xb0, xb1)
        return carry

    carry = lax.fori_loop(0, _N // 2, pair_step2, (zero, zero, zero, zero))
    wait_slot(ya0, ya1, xa0, xa1, semA)
    sx0, sx1, sy0, sy1 = carry

    ost[pl.ds(0, _LAN)] = sx0
    ost[pl.ds(_LAN, _LAN)] = sx1
    ost[pl.ds(2 * _LAN, _LAN)] = sy0
    ost[pl.ds(3 * _LAN, _LAN)] = sy1
    pltpu.sync_copy(ost, out.at[k])


def kernel(boundaries, edgemaps, boundary_lengths, edgemaps_len):
    bx = boundaries[..., 0].reshape(_N * _L)
    by = boundaries[..., 1].reshape(_N * _L)
    ex = edgemaps[..., 0].reshape(_N * _M)
    ey = edgemaps[..., 1].reshape(_N * _M)
    xl = boundary_lengths.reshape(_N).astype(jnp.int32)
    yl = edgemaps_len.reshape(_N).astype(jnp.int32)
    o = _chamfer_sc(bx, by, ex, ey, xl, yl)
    sx = o[:, : 2 * _LAN].sum(axis=0)
    sy = o[:, 2 * _LAN :].sum(axis=0)
    xlf = xl.astype(jnp.float32)
    ylf = yl.astype(jnp.float32)
    loss = sx / xlf + sy / ylf
    return loss.reshape(_B, _P).mean(axis=1) * 10.0
